# Initial kernel scaffold; baseline (speedup 1.0000x reference)
import jax
import jax.numpy as jnp
from jax.experimental import pallas as pl


def kernel(inputs, paddings, table):
    raise NotImplementedError("write your pallas kernel here")


# speedup vs baseline: 1.5113x; 1.5113x over previous
import jax
import jax.numpy as jnp
from jax.experimental import pallas as pl
from jax.experimental.pallas import tpu as pltpu

B, S, P, F = 4, 2048, 16, 128
D = P * F
ROWS = B * S
BLK = 256


def _body(pad_ref, tab_ref, in_ref, out_ref):
    pads = pad_ref[...]
    counts = P - jnp.sum(pads, axis=1)
    onehot = (counts[:, None] == jax.lax.broadcasted_iota(jnp.int32, (BLK, P + 1), 1)
              ).astype(jnp.float32)
    measured = jnp.dot(onehot, tab_ref[...],
                       preferred_element_type=jnp.float32)
    out_ref[...] = measured + in_ref[...]


def kernel(inputs, paddings, table):
    flat_in = inputs.reshape(ROWS, D)
    flat_pad = paddings.reshape(ROWS, P)
    out = pl.pallas_call(
        _body,
        grid=(ROWS // BLK,),
        in_specs=[
            pl.BlockSpec((BLK, P), lambda i: (i, 0)),
            pl.BlockSpec((P + 1, D), lambda i: (0, 0)),
            pl.BlockSpec((BLK, D), lambda i: (i, 0)),
        ],
        out_specs=pl.BlockSpec((BLK, D), lambda i: (i, 0)),
        out_shape=jax.ShapeDtypeStruct((ROWS, D), jnp.float32),
    )(flat_pad, table, flat_in)
    return out.reshape(B, S, D)


# speedup vs baseline: 1.5900x; 1.0521x over previous
import jax
import jax.numpy as jnp
from jax.experimental import pallas as pl
from jax.experimental.pallas import tpu as pltpu

B, S, P, F = 4, 2048, 16, 128
D = P * F
ROWS = B * S
BLK = 512


def _body(pad_ref, tab_ref, in_ref, out_ref):
    pads = pad_ref[...]
    counts = P - jnp.sum(pads, axis=1)
    onehot = (counts[:, None] == jax.lax.broadcasted_iota(jnp.int32, (BLK, P + 1), 1)
              ).astype(jnp.float32)
    measured = jnp.dot(onehot, tab_ref[...],
                       preferred_element_type=jnp.float32)
    out_ref[...] = measured + in_ref[...]


def kernel(inputs, paddings, table):
    flat_in = inputs.reshape(ROWS, D)
    flat_pad = paddings.reshape(ROWS, P)
    out = pl.pallas_call(
        _body,
        grid=(ROWS // BLK,),
        in_specs=[
            pl.BlockSpec((BLK, P), lambda i: (i, 0)),
            pl.BlockSpec((P + 1, D), lambda i: (0, 0)),
            pl.BlockSpec((BLK, D), lambda i: (i, 0)),
        ],
        out_specs=pl.BlockSpec((BLK, D), lambda i: (i, 0)),
        out_shape=jax.ShapeDtypeStruct((ROWS, D), jnp.float32),
    )(flat_pad, table, flat_in)
    return out.reshape(B, S, D)


# speedup vs baseline: 1.5947x; 1.0029x over previous
import jax
import jax.numpy as jnp
from jax.experimental import pallas as pl
from jax.experimental.pallas import tpu as pltpu

B, S, P, F = 4, 2048, 16, 128
D = P * F
ROWS = B * S
BLK = 1024


def _body(pad_ref, tab_ref, in_ref, out_ref):
    pads = pad_ref[...]
    counts = P - jnp.sum(pads, axis=1)
    onehot = (counts[:, None] == jax.lax.broadcasted_iota(jnp.int32, (BLK, P + 1), 1)
              ).astype(jnp.float32)
    measured = jnp.dot(onehot, tab_ref[...],
                       preferred_element_type=jnp.float32)
    out_ref[...] = measured + in_ref[...]


def kernel(inputs, paddings, table):
    flat_in = inputs.reshape(ROWS, D)
    flat_pad = paddings.reshape(ROWS, P)
    out = pl.pallas_call(
        _body,
        grid=(ROWS // BLK,),
        in_specs=[
            pl.BlockSpec((BLK, P), lambda i: (i, 0)),
            pl.BlockSpec((P + 1, D), lambda i: (0, 0)),
            pl.BlockSpec((BLK, D), lambda i: (i, 0)),
        ],
        out_specs=pl.BlockSpec((BLK, D), lambda i: (i, 0)),
        out_shape=jax.ShapeDtypeStruct((ROWS, D), jnp.float32),
    )(flat_pad, table, flat_in)
    return out.reshape(B, S, D)
